# Initial kernel scaffold; baseline (speedup 1.0000x reference)
import functools

import jax
import jax.numpy as jnp
from jax import lax
from jax.experimental import pallas as pl
from jax.experimental.pallas import tpu as pltpu
from jax.experimental.pallas import tpu_sc as plsc

N = 10000
E = 320000
D = 128
G = 64
NC = 2
NS = 16
NW = NC * NS
L = 16

N_PAD = 10240
ROWS = N_PAD // NS
K = 128
CH = 80
EPT = CH * K
E_PAD = NW * EPT
BM = 512
NB = N_PAD // BM

_mesh = plsc.VectorSubcoreMesh(core_axis_name="c", subcore_axis_name="s")


@functools.partial(
    pl.kernel,
    out_type=jax.ShapeDtypeStruct((NC, N_PAD), jnp.float32),
    mesh=_mesh,
    scratch_types=[
        pltpu.VMEM((CH, K), jnp.int32),
        pltpu.VMEM((K,), jnp.float32),
        pltpu.VMEM((ROWS,), jnp.float32),
        pltpu.VMEM_SHARED((N_PAD,), jnp.float32),
    ],
)
def _sc_degree(dst_hbm, deg2_hbm, dstv, onesv, zv, degsp):
    c = lax.axis_index("c")
    s = lax.axis_index("s")
    w = c * NS + s

    one = jnp.ones((L,), jnp.float32)
    zero = jnp.zeros((L,), jnp.float32)
    for i in range(K // L):
        onesv[pl.ds(i * L, L)] = one

    def zbody(i, _):
        zv[pl.ds(i * L, L)] = zero
        return 0

    lax.fori_loop(0, ROWS // L, zbody, 0)
    pltpu.sync_copy(zv, degsp.at[pl.ds(s * ROWS, ROWS)])
    pltpu.sync_copy(dst_hbm.at[w], dstv)
    plsc.subcore_barrier()

    def body(j, _):
        pltpu.sync_copy(onesv, degsp.at[dstv.at[j]], add=True)
        return 0

    lax.fori_loop(0, CH, body, 0)
    plsc.subcore_barrier()
    pltpu.sync_copy(degsp.at[pl.ds(s * ROWS, ROWS)],
                    deg2_hbm.at[c].at[pl.ds(s * ROWS, ROWS)])


@functools.partial(
    pl.kernel,
    out_type=jax.ShapeDtypeStruct((NC, N_PAD, D), jnp.float32),
    mesh=_mesh,
    scratch_types=[
        pltpu.VMEM((CH, K), jnp.int32),
        pltpu.VMEM((CH, K), jnp.int32),
        pltpu.VMEM((2, K, D), jnp.float32),
        pltpu.VMEM_SHARED((N_PAD, D), jnp.float32),
        pltpu.SemaphoreType.DMA,
        pltpu.SemaphoreType.DMA,
        pltpu.SemaphoreType.DMA,
        pltpu.SemaphoreType.DMA,
    ],
)
def _sc_propagate(g_hbm, src_hbm, dst_hbm, t_hbm,
                  srcv, dstv, rows, agg, sg0, sg1, ss0, ss1):
    c = lax.axis_index("c")
    s = lax.axis_index("s")
    w = c * NS + s

    zero = jnp.zeros((L,), jnp.float32)

    def zbody(r, _):
        for i in range(D // L):
            rows[0, r, pl.ds(i * L, L)] = zero
        return 0

    lax.fori_loop(0, K, zbody, 0)
    for i in range(ROWS // K):
        pltpu.sync_copy(rows.at[0],
                        agg.at[pl.ds(s * ROWS + i * K, K)])

    pltpu.sync_copy(src_hbm.at[w], srcv)
    pltpu.sync_copy(dst_hbm.at[w], dstv)

    pltpu.async_copy(g_hbm.at[srcv.at[0]], rows.at[0], sg0)
    plsc.subcore_barrier()

    def wait(sem, buf):
        pltpu.make_async_copy(g_hbm.at[pl.ds(0, K)], rows.at[buf], sem).wait()

    def body(gi, _):
        j0 = 2 * gi
        wait(sg0, 0)

        @pl.when(gi > 0)
        def _():
            wait(ss1, 1)

        pltpu.async_copy(g_hbm.at[srcv.at[j0 + 1]], rows.at[1], sg1)
        pltpu.async_copy(rows.at[0], agg.at[dstv.at[j0]], ss0, add=True)
        wait(sg1, 1)

        @pl.when(gi < CH // 2 - 1)
        def _():
            wait(ss0, 0)
            pltpu.async_copy(g_hbm.at[srcv.at[j0 + 2]], rows.at[0], sg0)

        pltpu.async_copy(rows.at[1], agg.at[dstv.at[j0 + 1]], ss1, add=True)
        return 0

    lax.fori_loop(0, CH // 2, body, 0)
    wait(ss0, 0)
    wait(ss1, 1)
    plsc.subcore_barrier()
    pltpu.sync_copy(agg.at[pl.ds(s * ROWS, ROWS)],
                    t_hbm.at[c].at[pl.ds(s * ROWS, ROWS)])


def _k1_body(x_ref, w_ref, deg_ref, g_ref, dinv_ref):
    deg = deg_ref[0] + deg_ref[1] + 1.0
    dv = lax.rsqrt(deg)
    dinv_ref[...] = dv
    h = jnp.dot(x_ref[...], w_ref[...], preferred_element_type=jnp.float32)
    g_ref[...] = h * dv


def _layer_body(relu, t_ref, g_ref, dinv_ref, b_ref, w_ref, out_ref):
    dv = dinv_ref[...]
    h = (t_ref[0] + t_ref[1] + g_ref[...]) * dv + b_ref[...]
    if relu:
        h = jnp.maximum(h, 0.0)
    out_ref[...] = jnp.dot(
        h, w_ref[...], preferred_element_type=jnp.float32) * dv


def _k4_body(t_ref, g_ref, dinv_ref, b_ref, batch_ref, wl_ref, bl_ref,
             out_ref, acc, cnt):
    i = pl.program_id(0)

    @pl.when(i == 0)
    def _():
        acc[...] = jnp.zeros_like(acc)
        cnt[...] = jnp.zeros_like(cnt)

    h = (t_ref[0] + t_ref[1] + g_ref[...]) * dinv_ref[...] + b_ref[...]
    gids = lax.broadcasted_iota(jnp.int32, (G, 1), 0)
    p = (batch_ref[...] == gids).astype(jnp.float32)
    acc[...] += jnp.dot(p, h, preferred_element_type=jnp.float32)
    cnt[...] += jnp.dot(p, jnp.ones((BM, D), jnp.float32),
                        preferred_element_type=jnp.float32)

    @pl.when(i == NB - 1)
    def _():
        pooled = acc[...] / jnp.maximum(cnt[...], 1.0)
        out_ref[...] = jnp.dot(
            pooled, wl_ref[...], preferred_element_type=jnp.float32
        ) + bl_ref[...]


_SPEC_ROWS = pl.BlockSpec((BM, D), lambda i: (i, 0))
_SPEC_T = pl.BlockSpec((NC, BM, D), lambda i: (0, i, 0))
_SPEC_COL = pl.BlockSpec((BM, 1), lambda i: (i, 0))
_SPEC_W = pl.BlockSpec((D, D), lambda i: (0, 0))
_SPEC_B = pl.BlockSpec((1, D), lambda i: (0, 0))

_k1 = pl.pallas_call(
    _k1_body,
    grid=(NB,),
    in_specs=[_SPEC_ROWS, _SPEC_W,
              pl.BlockSpec((NC, BM, 1), lambda i: (0, i, 0))],
    out_specs=(_SPEC_ROWS, _SPEC_COL),
    out_shape=(jax.ShapeDtypeStruct((N_PAD, D), jnp.float32),
               jax.ShapeDtypeStruct((N_PAD, 1), jnp.float32)),
)

_k2 = pl.pallas_call(
    functools.partial(_layer_body, True),
    grid=(NB,),
    in_specs=[_SPEC_T, _SPEC_ROWS, _SPEC_COL, _SPEC_B, _SPEC_W],
    out_specs=_SPEC_ROWS,
    out_shape=jax.ShapeDtypeStruct((N_PAD, D), jnp.float32),
)

_k4 = pl.pallas_call(
    _k4_body,
    grid=(NB,),
    in_specs=[_SPEC_T, _SPEC_ROWS, _SPEC_COL, _SPEC_B,
              pl.BlockSpec((G, BM), lambda i: (0, i)),
              pl.BlockSpec((D, 1), lambda i: (0, 0)),
              pl.BlockSpec((1, 1), lambda i: (0, 0))],
    out_specs=pl.BlockSpec((G, 1), lambda i: (0, 0)),
    out_shape=jax.ShapeDtypeStruct((G, 1), jnp.float32),
    scratch_shapes=[pltpu.VMEM((G, D), jnp.float32),
                    pltpu.VMEM((G, D), jnp.float32)],
)


def kernel(x, edge_index, batch, W1, b1, W2, b2, W3, b3, Wl, bl):
    x_pad = jnp.pad(x.astype(jnp.float32), ((0, N_PAD - N), (0, 0)))

    pe = E_PAD - E
    pad_idx = N + (jnp.arange(pe, dtype=jnp.int32) % (N_PAD - N))
    src = jnp.concatenate([edge_index[0], pad_idx]).reshape(NW, CH, K)
    dst = jnp.concatenate([edge_index[1], pad_idx]).reshape(NW, CH, K)

    batch_row = jnp.pad(batch, (0, N_PAD - N),
                        constant_values=G).reshape(1, N_PAD)

    deg2 = _sc_degree(dst).reshape(NC, N_PAD, 1)
    g1, dinv = _k1(x_pad, W1, deg2)
    t1 = _sc_propagate(g1, src, dst)
    g2 = _k2(t1, g1, dinv, b1.reshape(1, D), W2)
    t2 = _sc_propagate(g2, src, dst)
    g3 = _k2(t2, g2, dinv, b2.reshape(1, D), W3)
    t3 = _sc_propagate(g3, src, dst)
    return _k4(t3, g3, dinv, b3.reshape(1, D), batch_row,
               Wl, bl.reshape(1, 1))


# speedup vs baseline: 24.1200x; 24.1200x over previous
import functools

import jax
import jax.numpy as jnp
from jax import lax
from jax.experimental import pallas as pl
from jax.experimental.pallas import tpu as pltpu
from jax.experimental.pallas import tpu_sc as plsc

N = 10000
E = 320000
D = 128
G = 64
NC = 2
NS = 16
NW = NC * NS
L = 16

N_PAD = 10240
ROWS = N_PAD // NS
K = 128
CH = 80
EPT = CH * K
E_PAD = NW * EPT
BM = 512
NB = N_PAD // BM

_mesh = plsc.VectorSubcoreMesh(core_axis_name="c", subcore_axis_name="s",
                               num_cores=NC, num_subcores=NS)


@functools.partial(
    pl.kernel,
    out_type=jax.ShapeDtypeStruct((NC, N_PAD), jnp.float32),
    mesh=_mesh,
    scratch_types=[
        pltpu.VMEM((CH, K), jnp.int32),
        pltpu.VMEM((K,), jnp.float32),
        pltpu.VMEM((ROWS,), jnp.float32),
        pltpu.VMEM_SHARED((N_PAD,), jnp.float32),
    ],
)
def _sc_degree(dst_hbm, deg2_hbm, dstv, onesv, zv, degsp):
    c = lax.axis_index("c")
    s = lax.axis_index("s")
    w = c * NS + s

    one = jnp.ones((L,), jnp.float32)
    zero = jnp.zeros((L,), jnp.float32)
    for i in range(K // L):
        onesv[pl.ds(i * L, L)] = one

    def zbody(i, _):
        zv[pl.ds(i * L, L)] = zero
        return 0

    lax.fori_loop(0, ROWS // L, zbody, 0)
    pltpu.sync_copy(zv, degsp.at[pl.ds(s * ROWS, ROWS)])
    pltpu.sync_copy(dst_hbm.at[w], dstv)
    plsc.subcore_barrier()

    def body(j, _):
        pltpu.sync_copy(onesv, degsp.at[dstv.at[j]], add=True)
        return 0

    lax.fori_loop(0, CH, body, 0)
    plsc.subcore_barrier()
    pltpu.sync_copy(degsp.at[pl.ds(s * ROWS, ROWS)],
                    deg2_hbm.at[c].at[pl.ds(s * ROWS, ROWS)])


@functools.partial(
    pl.kernel,
    out_type=jax.ShapeDtypeStruct((NC, N_PAD, D), jnp.float32),
    mesh=_mesh,
    scratch_types=[
        pltpu.VMEM((CH // 2, K), jnp.int32),
        pltpu.VMEM((CH // 2, K), jnp.int32),
        pltpu.VMEM((2, K, D), jnp.float32),
        pltpu.VMEM_SHARED((N_PAD, D), jnp.float32),
        pltpu.SemaphoreType.DMA,
        pltpu.SemaphoreType.DMA,
        pltpu.SemaphoreType.DMA,
        pltpu.SemaphoreType.DMA,
    ],
)
def _sc_propagate(g_hbm, src_hbm, dst_hbm, t_hbm,
                  srcv, dstv, rows, agg, sg0, sg1, ss0, ss1):
    c = lax.axis_index("c")
    s = lax.axis_index("s")
    w = c * NS + s

    zero = jnp.zeros((L,), jnp.float32)

    def zbody(r, _):
        for i in range(D // L):
            rows[0, r, pl.ds(i * L, L)] = zero
        return 0

    lax.fori_loop(0, K, zbody, 0)
    for i in range(ROWS // K):
        pltpu.sync_copy(rows.at[0],
                        agg.at[pl.ds(s * ROWS + i * K, K)])

    plsc.subcore_barrier()

    def wait(sem, buf):
        pltpu.make_async_copy(g_hbm.at[pl.ds(0, K)], rows.at[buf], sem).wait()

    half = CH // 2
    for ph in range(2):
        pltpu.sync_copy(src_hbm.at[w].at[pl.ds(ph * half, half)], srcv)
        pltpu.sync_copy(dst_hbm.at[w].at[pl.ds(ph * half, half)], dstv)
        pltpu.async_copy(g_hbm.at[srcv.at[0]], rows.at[0], sg0)

        def body(gi, _):
            j0 = 2 * gi
            wait(sg0, 0)

            @pl.when(gi > 0)
            def _():
                wait(ss1, 1)

            pltpu.async_copy(g_hbm.at[srcv.at[j0 + 1]], rows.at[1], sg1)
            pltpu.async_copy(rows.at[0], agg.at[dstv.at[j0]], ss0, add=True)
            wait(sg1, 1)

            @pl.when(gi < half // 2 - 1)
            def _():
                wait(ss0, 0)
                pltpu.async_copy(g_hbm.at[srcv.at[j0 + 2]], rows.at[0], sg0)

            pltpu.async_copy(rows.at[1], agg.at[dstv.at[j0 + 1]], ss1, add=True)
            return 0

        lax.fori_loop(0, half // 2, body, 0)
        wait(ss0, 0)
        wait(ss1, 1)
    plsc.subcore_barrier()
    pltpu.sync_copy(agg.at[pl.ds(s * ROWS, ROWS)],
                    t_hbm.at[c].at[pl.ds(s * ROWS, ROWS)])


def _k1_body(x_ref, w_ref, deg_ref, g_ref, dinv_ref):
    deg = deg_ref[0] + deg_ref[1] + 1.0
    dv = lax.rsqrt(deg)
    dinv_ref[...] = dv
    h = jnp.dot(x_ref[...], w_ref[...], preferred_element_type=jnp.float32)
    g_ref[...] = h * dv


def _layer_body(relu, t_ref, g_ref, dinv_ref, b_ref, w_ref, out_ref):
    dv = dinv_ref[...]
    h = (t_ref[0] + t_ref[1] + g_ref[...]) * dv + b_ref[...]
    if relu:
        h = jnp.maximum(h, 0.0)
    out_ref[...] = jnp.dot(
        h, w_ref[...], preferred_element_type=jnp.float32) * dv


def _k4_body(t_ref, g_ref, dinv_ref, b_ref, batch_ref, wl_ref, bl_ref,
             out_ref, acc, cnt):
    i = pl.program_id(0)

    @pl.when(i == 0)
    def _():
        acc[...] = jnp.zeros_like(acc)
        cnt[...] = jnp.zeros_like(cnt)

    h = (t_ref[0] + t_ref[1] + g_ref[...]) * dinv_ref[...] + b_ref[...]
    gids = lax.broadcasted_iota(jnp.int32, (G, 1), 0)
    p = (batch_ref[...] == gids).astype(jnp.float32)
    acc[...] += jnp.dot(p, h, preferred_element_type=jnp.float32)
    cnt[...] += jnp.dot(p, jnp.ones((BM, D), jnp.float32),
                        preferred_element_type=jnp.float32)

    @pl.when(i == NB - 1)
    def _():
        pooled = acc[...] / jnp.maximum(cnt[...], 1.0)
        out_ref[...] = jnp.dot(
            pooled, wl_ref[...], preferred_element_type=jnp.float32
        ) + bl_ref[...]


_SPEC_ROWS = pl.BlockSpec((BM, D), lambda i: (i, 0))
_SPEC_T = pl.BlockSpec((NC, BM, D), lambda i: (0, i, 0))
_SPEC_COL = pl.BlockSpec((BM, 1), lambda i: (i, 0))
_SPEC_W = pl.BlockSpec((D, D), lambda i: (0, 0))
_SPEC_B = pl.BlockSpec((1, D), lambda i: (0, 0))

_k1 = pl.pallas_call(
    _k1_body,
    grid=(NB,),
    in_specs=[_SPEC_ROWS, _SPEC_W,
              pl.BlockSpec((NC, BM, 1), lambda i: (0, i, 0))],
    out_specs=(_SPEC_ROWS, _SPEC_COL),
    out_shape=(jax.ShapeDtypeStruct((N_PAD, D), jnp.float32),
               jax.ShapeDtypeStruct((N_PAD, 1), jnp.float32)),
)

_k2 = pl.pallas_call(
    functools.partial(_layer_body, True),
    grid=(NB,),
    in_specs=[_SPEC_T, _SPEC_ROWS, _SPEC_COL, _SPEC_B, _SPEC_W],
    out_specs=_SPEC_ROWS,
    out_shape=jax.ShapeDtypeStruct((N_PAD, D), jnp.float32),
)

_k4 = pl.pallas_call(
    _k4_body,
    grid=(NB,),
    in_specs=[_SPEC_T, _SPEC_ROWS, _SPEC_COL, _SPEC_B,
              pl.BlockSpec((1, BM), lambda i: (0, i)),
              pl.BlockSpec((D, 1), lambda i: (0, 0)),
              pl.BlockSpec((1, 1), lambda i: (0, 0))],
    out_specs=pl.BlockSpec((G, 1), lambda i: (0, 0)),
    out_shape=jax.ShapeDtypeStruct((G, 1), jnp.float32),
    scratch_shapes=[pltpu.VMEM((G, D), jnp.float32),
                    pltpu.VMEM((G, D), jnp.float32)],
)


def kernel(x, edge_index, batch, W1, b1, W2, b2, W3, b3, Wl, bl):
    x_pad = jnp.pad(x.astype(jnp.float32), ((0, N_PAD - N), (0, 0)))

    pe = E_PAD - E
    pad_idx = N + (jnp.arange(pe, dtype=jnp.int32) % (N_PAD - N))
    src = jnp.concatenate([edge_index[0], pad_idx]).reshape(NW, CH, K)
    dst = jnp.concatenate([edge_index[1], pad_idx]).reshape(NW, CH, K)

    batch_row = jnp.pad(batch, (0, N_PAD - N),
                        constant_values=G).reshape(1, N_PAD)

    deg2 = _sc_degree(dst).reshape(NC, N_PAD, 1)
    g1, dinv = _k1(x_pad, W1, deg2)
    t1 = _sc_propagate(g1, src, dst)
    g2 = _k2(t1, g1, dinv, b1.reshape(1, D), W2)
    t2 = _sc_propagate(g2, src, dst)
    g3 = _k2(t2, g2, dinv, b2.reshape(1, D), W3)
    t3 = _sc_propagate(g3, src, dst)
    return _k4(t3, g3, dinv, b3.reshape(1, D), batch_row,
               Wl, bl.reshape(1, 1))
